# Initial kernel scaffold; baseline (speedup 1.0000x reference)
import jax
import jax.numpy as jnp
from jax.experimental import pallas as pl


def kernel(x, edge_index, batch, W1l, b1l, W1r, W2l, b2l, W2r, Wg1, bg1, Wg2, bg2, Wo, bo):
    raise NotImplementedError("write your pallas kernel here")


# speedup vs baseline: 4.3120x; 4.3120x over previous
import functools

import jax
import jax.numpy as jnp
from jax import lax
from jax.experimental import pallas as pl
from jax.experimental.pallas import tpu as pltpu
from jax.experimental.pallas import tpu_sc as plsc

N = 10000
E = 320000
D = 128
G = 128
DA = 144
NC = 2
NS = 16
NW = NC * NS
C = 128
KCH = 79
EPAD = NW * KCH * C
R = 10112
RPT = R // NS
TRASH = 10000
BR = 2528
F32 = jnp.float32


def _agg_body(table, srcp, dstp, zrows, acc_out, src_v, dst_v, rows_v, acc_sh, sem):
    c = lax.axis_index("c")
    s = lax.axis_index("s")
    wid = s * NC + c
    rbase = s * RPT
    pltpu.sync_copy(zrows, acc_sh.at[pl.ds(rbase, RPT)])
    pltpu.sync_copy(srcp.at[wid], src_v)
    pltpu.sync_copy(dstp.at[wid], dst_v)
    plsc.subcore_barrier()

    def step(j, carry):
        pltpu.async_copy(table.at[src_v.at[j]], rows_v, sem).wait()
        pltpu.sync_copy(rows_v, acc_sh.at[dst_v.at[j]], add=True)
        return carry

    lax.fori_loop(0, KCH, step, 0)
    plsc.subcore_barrier()
    pltpu.sync_copy(acc_sh.at[pl.ds(rbase, RPT)],
                    acc_out.at[c, pl.ds(rbase, RPT)])


_sc_aggregate = pl.kernel(
    _agg_body,
    out_type=jax.ShapeDtypeStruct((NC, R, DA), F32),
    mesh=plsc.VectorSubcoreMesh(core_axis_name="c", subcore_axis_name="s",
                                num_cores=NC, num_subcores=NS),
    scratch_types=[
        pltpu.VMEM((KCH, C), jnp.int32),
        pltpu.VMEM((KCH, C), jnp.int32),
        pltpu.VMEM((C, DA), F32),
        pltpu.VMEM_SHARED((R, DA), F32),
        pltpu.SemaphoreType.DMA,
    ],
    compiler_params=pltpu.CompilerParams(use_tc_tiling_on_sc=False),
)


def _dense_body(acc_ref, xin_ref, wl_ref, wr_ref, b_ref, out_ref, *, relu, aug):
    acc = acc_ref[0] + acc_ref[1]
    cnt = jnp.maximum(acc[:, D:D + 1], 1.0)
    mean = acc[:, :D] / cnt
    xin = xin_ref[...][:, :D]
    h = (jnp.dot(mean, wl_ref[...], preferred_element_type=F32)
         + jnp.dot(xin, wr_ref[...], preferred_element_type=F32)
         + b_ref[...])
    if relu:
        h = jnp.maximum(h, 0.0)
    if aug:
        pad = jnp.concatenate(
            [jnp.ones((BR, 1), F32), jnp.zeros((BR, DA - D - 1), F32)], axis=1)
        out_ref[...] = jnp.concatenate([h, pad], axis=1)
    else:
        out_ref[...] = h


def _dense_layer(acc, xin, wlT, wrT, b2d, *, relu, aug):
    width = DA if aug else D
    return pl.pallas_call(
        functools.partial(_dense_body, relu=relu, aug=aug),
        grid=(R // BR,),
        in_specs=[
            pl.BlockSpec((NC, BR, DA), lambda i: (0, i, 0)),
            pl.BlockSpec((BR, DA), lambda i: (i, 0)),
            pl.BlockSpec((D, D), lambda i: (0, 0)),
            pl.BlockSpec((D, D), lambda i: (0, 0)),
            pl.BlockSpec((1, D), lambda i: (0, 0)),
        ],
        out_specs=pl.BlockSpec((BR, width), lambda i: (i, 0)),
        out_shape=jax.ShapeDtypeStruct((R, width), F32),
    )(acc, xin, wlT, wrT, b2d)


def _pool_head_body(h2_ref, bt_ref, wg1_ref, bg1_ref, wg2_ref, bg2_ref,
                    wo_ref, bo_ref, out_ref):
    run = h2_ref[...]
    b = bt_ref[...]
    d = 1
    while d < R:
        bsh = jnp.concatenate([jnp.full((d, 1), -1, jnp.int32), b[:R - d]], axis=0)
        rsh = jnp.concatenate([run[:d], run[:R - d]], axis=0)
        run = jnp.where(b == bsh, jnp.maximum(run, rsh), run)
        d *= 2
    nxt = jnp.concatenate([b[1:], jnp.full((1, 1), -2, jnp.int32)], axis=0)
    last = (b != nxt)
    onehot = ((b == lax.broadcasted_iota(jnp.int32, (R, G), 1)) & last).astype(F32)
    g = lax.dot_general(onehot, run, (((0,), (0,)), ((), ())),
                        preferred_element_type=F32)
    g = jnp.maximum(jnp.dot(g, wg1_ref[...], preferred_element_type=F32)
                    + bg1_ref[...], 0.0)
    g = jnp.maximum(jnp.dot(g, wg2_ref[...], preferred_element_type=F32)
                    + bg2_ref[...], 0.0)
    out_ref[...] = jnp.dot(g, wo_ref[...], preferred_element_type=F32) + bo_ref[...]


def _pool_head(h2, bt, wg1T, bg1, wg2T, bg2, woT, bo2d):
    return pl.pallas_call(
        _pool_head_body,
        out_shape=jax.ShapeDtypeStruct((G, 1), F32),
    )(h2, bt, wg1T, bg1, wg2T, bg2, woT, bo2d)


def kernel(x, edge_index, batch, W1l, b1l, W1r, W2l, b2l, W2r,
           Wg1, bg1, Wg2, bg2, Wo, bo):
    src = edge_index[0].astype(jnp.int32)
    dst = edge_index[1].astype(jnp.int32)
    srcp = jnp.concatenate(
        [src, jnp.zeros((EPAD - E,), jnp.int32)]).reshape(NW, KCH, C)
    dstp = jnp.concatenate(
        [dst, jnp.full((EPAD - E,), TRASH, jnp.int32)]).reshape(NW, KCH, C)

    xa = jnp.concatenate(
        [x, jnp.ones((N, 1), F32), jnp.zeros((N, DA - D - 1), F32)], axis=1)
    xa = jnp.concatenate([xa, jnp.zeros((R - N, DA), F32)], axis=0)
    zrows = jnp.zeros((RPT, DA), F32)

    bt = jnp.concatenate(
        [batch.astype(jnp.int32), jnp.full((R - N,), G, jnp.int32)]
    ).reshape(R, 1)

    acc1 = _sc_aggregate(xa, srcp, dstp, zrows)
    h1a = _dense_layer(acc1, xa, W1l.T, W1r.T, b1l.reshape(1, D),
                       relu=True, aug=True)
    acc2 = _sc_aggregate(h1a, srcp, dstp, zrows)
    h2 = _dense_layer(acc2, h1a, W2l.T, W2r.T, b2l.reshape(1, D),
                      relu=False, aug=False)
    return _pool_head(h2, bt, Wg1.T, bg1.reshape(1, D), Wg2.T,
                      bg2.reshape(1, D), Wo.T, bo.reshape(1, 1))
